# Initial kernel scaffold; baseline (speedup 1.0000x reference)
import jax
import jax.numpy as jnp
from jax.experimental import pallas as pl


def kernel(x, table, gamma, beta):
    raise NotImplementedError("write your pallas kernel here")


# speedup vs baseline: 3.4235x; 3.4235x over previous
import functools

import jax
import jax.numpy as jnp
from jax import lax
from jax.experimental import pallas as pl
from jax.experimental.pallas import tpu as pltpu
from jax.experimental.pallas import tpu_sc as plsc

EPS = 1e-05


def _ln_table_body(table_ref, gamma_ref, beta_ref, out_ref):
    t = table_ref[...]
    mean = jnp.mean(t, axis=-1, keepdims=True)
    var = jnp.mean(jnp.square(t - mean), axis=-1, keepdims=True)
    out_ref[...] = (t - mean) / jnp.sqrt(var + EPS) * gamma_ref[...] + beta_ref[...]


def _normalize_table(table, gamma, beta):
    V, D = table.shape
    return pl.pallas_call(
        _ln_table_body,
        out_shape=jax.ShapeDtypeStruct((V, D), jnp.float32),
    )(table, gamma.reshape(1, D), beta.reshape(1, D))


@functools.lru_cache(maxsize=None)
def _make_sc_gather(V, D, N):
    info = plsc.get_sparse_core_info()
    NC, NS = info.num_cores, info.num_subcores
    NW = NC * NS
    assert N % NW == 0
    n_per_w = N // NW
    CHUNK = 128
    assert n_per_w % CHUNK == 0
    n_chunks = n_per_w // CHUNK

    mesh = plsc.VectorSubcoreMesh(core_axis_name="c", subcore_axis_name="s")

    @functools.partial(
        pl.kernel,
        mesh=mesh,
        out_type=jax.ShapeDtypeStruct((N, D), jnp.float32),
        scratch_types=[
            pltpu.VMEM((CHUNK,), jnp.int32),
            pltpu.VMEM((CHUNK, D), jnp.float32),
            pltpu.SemaphoreType.DMA,
        ],
    )
    def gather_kernel(table_hbm, idx_hbm, out_hbm, idx_v, rows_v, sem):
        wid = lax.axis_index("s") * NC + lax.axis_index("c")
        base = wid * n_per_w

        def body(c, carry):
            off = base + c * CHUNK
            pltpu.sync_copy(idx_hbm.at[pl.ds(off, CHUNK)], idx_v)
            pltpu.async_copy(table_hbm.at[idx_v], rows_v, sem).wait()
            pltpu.sync_copy(rows_v, out_hbm.at[pl.ds(off, CHUNK)])
            return carry

        lax.fori_loop(0, n_chunks, body, 0)

    return gather_kernel


def kernel(x, table, gamma, beta):
    B, L = x.shape
    V, D = table.shape
    normed = _normalize_table(table, gamma, beta)
    idx = x.reshape(-1).astype(jnp.int32)
    out = _make_sc_gather(V, D, B * L)(normed, idx)
    return out.reshape(B, L, D)


# speedup vs baseline: 3.7759x; 1.1029x over previous
import functools

import jax
import jax.numpy as jnp
from jax import lax
from jax.experimental import pallas as pl
from jax.experimental.pallas import tpu as pltpu
from jax.experimental.pallas import tpu_sc as plsc

EPS = 1e-05


def _ln_table_body(table_ref, gamma_ref, beta_ref, out_ref):
    t = table_ref[...]
    mean = jnp.mean(t, axis=-1, keepdims=True)
    var = jnp.mean(jnp.square(t - mean), axis=-1, keepdims=True)
    out_ref[...] = (t - mean) / jnp.sqrt(var + EPS) * gamma_ref[...] + beta_ref[...]


def _normalize_table(table, gamma, beta):
    V, D = table.shape
    return pl.pallas_call(
        _ln_table_body,
        out_shape=jax.ShapeDtypeStruct((V, D), jnp.float32),
    )(table, gamma.reshape(1, D), beta.reshape(1, D))


NBUF = 4


@functools.lru_cache(maxsize=None)
def _make_sc_gather(V, D, N):
    info = plsc.get_sparse_core_info()
    NC, NS = info.num_cores, info.num_subcores
    NW = NC * NS
    assert N % NW == 0
    n_per_w = N // NW
    CHUNK = 128
    assert n_per_w % (CHUNK * NBUF) == 0
    n_chunks = n_per_w // CHUNK
    n_groups = n_chunks // NBUF

    mesh = plsc.VectorSubcoreMesh(core_axis_name="c", subcore_axis_name="s")

    @functools.partial(
        pl.kernel,
        mesh=mesh,
        out_type=jax.ShapeDtypeStruct((N, D), jnp.float32),
        scratch_types=[
            pltpu.VMEM((n_chunks, CHUNK), jnp.int32),
        ]
        + [pltpu.VMEM((CHUNK, D), jnp.float32) for _ in range(NBUF)]
        + [pltpu.SemaphoreType.DMA for _ in range(2 * NBUF)],
    )
    def gather_kernel(table_hbm, idx_hbm, out_hbm, idx_v, *bufs_and_sems):
        bufs = bufs_and_sems[:NBUF]
        gsem = bufs_and_sems[NBUF:2 * NBUF]
        wsem = bufs_and_sems[2 * NBUF:]
        wid = lax.axis_index("s") * NC + lax.axis_index("c")
        base = wid * n_per_w

        pltpu.sync_copy(idx_hbm.at[wid], idx_v)

        for b in range(NBUF):
            pltpu.async_copy(table_hbm.at[idx_v.at[b]], bufs[b], gsem[b])

        def body(g, carry):
            for b in range(NBUF):
                c = NBUF * g + b
                pltpu.make_async_copy(table_hbm.at[idx_v.at[c]], bufs[b], gsem[b]).wait()
                cp = pltpu.async_copy(
                    bufs[b], out_hbm.at[pl.ds(base + c * CHUNK, CHUNK)], wsem[b]
                )
                cp.wait()
                pltpu.async_copy(table_hbm.at[idx_v.at[c + NBUF]], bufs[b], gsem[b])
            return carry

        lax.fori_loop(0, n_groups - 1, body, 0)

        for b in range(NBUF):
            c = NBUF * (n_groups - 1) + b
            pltpu.make_async_copy(table_hbm.at[idx_v.at[c]], bufs[b], gsem[b]).wait()
            pltpu.async_copy(
                bufs[b], out_hbm.at[pl.ds(base + c * CHUNK, CHUNK)], wsem[b]
            ).wait()

    return gather_kernel


def kernel(x, table, gamma, beta):
    B, L = x.shape
    V, D = table.shape
    normed = _normalize_table(table, gamma, beta)
    N = B * L
    info = plsc.get_sparse_core_info()
    NW = info.num_cores * info.num_subcores
    idx = x.reshape(NW, (N // NW) // 128, 128).astype(jnp.int32)
    out = _make_sc_gather(V, D, N)(normed, idx)
    return out.reshape(B, L, D)


# speedup vs baseline: 5.9128x; 1.5659x over previous
import functools

import jax
import jax.numpy as jnp
from jax import lax
from jax.experimental import pallas as pl
from jax.experimental.pallas import tpu as pltpu
from jax.experimental.pallas import tpu_sc as plsc

EPS = 1e-05


def _ln_table_body(table_ref, gamma_ref, beta_ref, out_ref):
    t = table_ref[...]
    mean = jnp.mean(t, axis=-1, keepdims=True)
    var = jnp.mean(jnp.square(t - mean), axis=-1, keepdims=True)
    out_ref[...] = (t - mean) / jnp.sqrt(var + EPS) * gamma_ref[...] + beta_ref[...]


def _normalize_table(table, gamma, beta):
    V, D = table.shape
    return pl.pallas_call(
        _ln_table_body,
        out_shape=jax.ShapeDtypeStruct((V, D), jnp.float32),
    )(table, gamma.reshape(1, D), beta.reshape(1, D))


NBUF = 4
EPC = 4


@functools.lru_cache(maxsize=None)
def _make_sc_gather(V, D, B, L):
    info = plsc.get_sparse_core_info()
    NC, NS = info.num_cores, info.num_subcores
    NW = NC * NS
    assert B % NW == 0
    e_per_w = B // NW
    CHUNK = EPC * L
    assert CHUNK <= 128
    assert e_per_w % (EPC * NBUF) == 0
    n_chunks = e_per_w // EPC
    n_groups = n_chunks // NBUF

    mesh = plsc.VectorSubcoreMesh(core_axis_name="c", subcore_axis_name="s")

    @functools.partial(
        pl.kernel,
        mesh=mesh,
        out_type=jax.ShapeDtypeStruct((B, L, D), jnp.float32),
        scratch_types=[
            pltpu.VMEM((n_chunks, CHUNK), jnp.int32),
        ]
        + [pltpu.VMEM((EPC * L, D), jnp.float32) for _ in range(NBUF)]
        + [pltpu.SemaphoreType.DMA for _ in range(2 * NBUF)],
    )
    def gather_kernel(table_hbm, idx_hbm, out_hbm, idx_v, *bufs_and_sems):
        bufs = bufs_and_sems[:NBUF]
        gsem = bufs_and_sems[NBUF:2 * NBUF]
        wsem = bufs_and_sems[2 * NBUF:]
        wid = lax.axis_index("s") * NC + lax.axis_index("c")
        ebase = wid * e_per_w

        pltpu.sync_copy(idx_hbm.at[wid], idx_v)

        for b in range(NBUF):
            pltpu.async_copy(table_hbm.at[idx_v.at[b]], bufs[b], gsem[b])

        def body(g, carry):
            for b in range(NBUF):
                c = NBUF * g + b
                pltpu.make_async_copy(table_hbm.at[idx_v.at[c]], bufs[b], gsem[b]).wait()
                pltpu.async_copy(
                    bufs[b].reshape(EPC, L, D),
                    out_hbm.at[pl.ds(ebase + c * EPC, EPC)],
                    wsem[b],
                ).wait()
                pltpu.async_copy(table_hbm.at[idx_v.at[c + NBUF]], bufs[b], gsem[b])
            return carry

        lax.fori_loop(0, n_groups - 1, body, 0)

        for b in range(NBUF):
            c = NBUF * (n_groups - 1) + b
            pltpu.make_async_copy(table_hbm.at[idx_v.at[c]], bufs[b], gsem[b]).wait()
            pltpu.async_copy(
                bufs[b].reshape(EPC, L, D),
                out_hbm.at[pl.ds(ebase + c * EPC, EPC)],
                wsem[b],
            ).wait()

    return gather_kernel


def kernel(x, table, gamma, beta):
    B, L = x.shape
    V, D = table.shape
    normed = _normalize_table(table, gamma, beta)
    info = plsc.get_sparse_core_info()
    NW = info.num_cores * info.num_subcores
    idx = x.reshape(NW, (B // NW) // EPC, EPC * L).astype(jnp.int32)
    return _make_sc_gather(V, D, B, L)(normed, idx)


# speedup vs baseline: 22.4415x; 3.7954x over previous
import functools

import jax
import jax.numpy as jnp
from jax import lax
from jax.experimental import pallas as pl
from jax.experimental.pallas import tpu as pltpu
from jax.experimental.pallas import tpu_sc as plsc

EPS = 1e-05


def _ln_table_body(table_ref, gamma_ref, beta_ref, out_ref):
    t = table_ref[...]
    mean = jnp.mean(t, axis=-1, keepdims=True)
    var = jnp.mean(jnp.square(t - mean), axis=-1, keepdims=True)
    out_ref[...] = (t - mean) / jnp.sqrt(var + EPS) * gamma_ref[...] + beta_ref[...]


def _normalize_table(table, gamma, beta):
    V, D = table.shape
    return pl.pallas_call(
        _ln_table_body,
        out_shape=jax.ShapeDtypeStruct((V, D), jnp.float32),
    )(table, gamma.reshape(1, D), beta.reshape(1, D))


NBUF = 4
CHUNK = 128


@functools.lru_cache(maxsize=None)
def _make_sc_gather(V, D, N):
    info = plsc.get_sparse_core_info()
    NC, NS = info.num_cores, info.num_subcores
    NW = NC * NS
    assert N % NW == 0
    n_per_w = N // NW
    assert n_per_w % (CHUNK * NBUF) == 0
    n_chunks = n_per_w // CHUNK
    n_groups = n_chunks // NBUF

    mesh = plsc.VectorSubcoreMesh(core_axis_name="c", subcore_axis_name="s")

    @functools.partial(
        pl.kernel,
        mesh=mesh,
        out_type=jax.ShapeDtypeStruct((N, D), jnp.float32),
        scratch_types=[
            pltpu.VMEM((n_chunks, CHUNK), jnp.int32),
            pltpu.VMEM_SHARED((V, D), jnp.float32),
        ]
        + [pltpu.VMEM((CHUNK, D), jnp.float32) for _ in range(NBUF)]
        + [pltpu.SemaphoreType.DMA for _ in range(2 * NBUF)],
    )
    def gather_kernel(table_hbm, idx_hbm, out_hbm, idx_v, table_sp, *bufs_and_sems):
        bufs = bufs_and_sems[:NBUF]
        gsem = bufs_and_sems[NBUF:2 * NBUF]
        wsem = bufs_and_sems[2 * NBUF:]
        sid = lax.axis_index("s")
        wid = sid * NC + lax.axis_index("c")
        base = wid * n_per_w

        @pl.when(sid == 0)
        def _stage_table():
            pltpu.sync_copy(table_hbm, table_sp)

        pltpu.sync_copy(idx_hbm.at[wid], idx_v)
        plsc.subcore_barrier()

        for b in range(NBUF):
            pltpu.async_copy(table_sp.at[idx_v.at[b]], bufs[b], gsem[b])

        def body(g, carry):
            for b in range(NBUF):
                c = NBUF * g + b
                pltpu.make_async_copy(table_sp.at[idx_v.at[c]], bufs[b], gsem[b]).wait()
                pltpu.async_copy(
                    bufs[b], out_hbm.at[pl.ds(base + c * CHUNK, CHUNK)], wsem[b]
                ).wait()
                pltpu.async_copy(table_sp.at[idx_v.at[c + NBUF]], bufs[b], gsem[b])
            return carry

        lax.fori_loop(0, n_groups - 1, body, 0)

        for b in range(NBUF):
            c = NBUF * (n_groups - 1) + b
            pltpu.make_async_copy(table_sp.at[idx_v.at[c]], bufs[b], gsem[b]).wait()
            pltpu.async_copy(
                bufs[b], out_hbm.at[pl.ds(base + c * CHUNK, CHUNK)], wsem[b]
            ).wait()

    return gather_kernel


def kernel(x, table, gamma, beta):
    B, L = x.shape
    V, D = table.shape
    normed = _normalize_table(table, gamma, beta)
    N = B * L
    info = plsc.get_sparse_core_info()
    NW = info.num_cores * info.num_subcores
    idx = x.T.reshape(NW, (N // NW) // CHUNK, CHUNK).astype(jnp.int32)
    out = _make_sc_gather(V, D, N)(normed, idx)
    return out.reshape(L, B, D).transpose(1, 0, 2)
